# Initial kernel scaffold; baseline (speedup 1.0000x reference)
import jax
import jax.numpy as jnp
from jax.experimental import pallas as pl


def kernel(input, segm_img):
    raise NotImplementedError("write your pallas kernel here")


# speedup vs baseline: 378.3055x; 378.3055x over previous
import functools

import jax
import jax.numpy as jnp
from jax import lax
from jax.experimental import pallas as pl
from jax.experimental.pallas import tpu as pltpu
from jax.experimental.pallas import tpu_sc as plsc

H = 2048
W = 2048
N = H * W
TABLE = 10000

_info = plsc.get_sparse_core_info()
NC = _info.num_cores
NS = _info.num_subcores
L = _info.num_lanes
NW = NC * NS
PER_W = N // NW
CHUNK = 16384
NCHUNK = PER_W // CHUNK
UNROLL = 8

_mesh = plsc.VectorSubcoreMesh(core_axis_name="c", subcore_axis_name="s")


@functools.partial(
    pl.kernel,
    mesh=_mesh,
    compiler_params=pltpu.CompilerParams(needs_layout_passes=False),
    out_type=jax.ShapeDtypeStruct((N,), jnp.float32),
    scratch_types=[
        pltpu.VMEM((TABLE,), jnp.float32),
        pltpu.VMEM((CHUNK,), jnp.int32),
        pltpu.VMEM((CHUNK,), jnp.float32),
    ],
)
def _superpixel_gather(table_hbm, segm_hbm, out_hbm, table_v, idx_v, out_v):
    wid = lax.axis_index("s") * NC + lax.axis_index("c")
    base = wid * PER_W
    pltpu.sync_copy(table_hbm, table_v)

    def chunk_body(ci, carry):
        off = base + ci * CHUNK
        pltpu.sync_copy(segm_hbm.at[pl.ds(off, CHUNK)], idx_v)

        def gstep(i, c):
            for u in range(UNROLL):
                s = (i * UNROLL + u) * L
                idx = idx_v[pl.ds(s, L)]
                out_v[pl.ds(s, L)] = plsc.load_gather(table_v, [idx])
            return c

        lax.fori_loop(0, CHUNK // (L * UNROLL), gstep, 0)
        pltpu.sync_copy(out_v, out_hbm.at[pl.ds(off, CHUNK)])
        return carry

    lax.fori_loop(0, NCHUNK, chunk_body, 0)


def kernel(input, segm_img):
    flat = segm_img.reshape(-1)
    out = _superpixel_gather(input, flat)
    return out.reshape(segm_img.shape)


# speedup vs baseline: 492.9036x; 1.3029x over previous
import functools

import jax
import jax.numpy as jnp
from jax import lax
from jax.experimental import pallas as pl
from jax.experimental.pallas import tpu as pltpu
from jax.experimental.pallas import tpu_sc as plsc

H = 2048
W = 2048
TABLE = 10000

_info = plsc.get_sparse_core_info()
NC = _info.num_cores
NS = _info.num_subcores
L = _info.num_lanes
NW = NC * NS
ROWS_W = H // NW
BR = 8
NBLK = ROWS_W // BR

_mesh = plsc.VectorSubcoreMesh(core_axis_name="c", subcore_axis_name="s")


@functools.partial(
    pl.kernel,
    mesh=_mesh,
    compiler_params=pltpu.CompilerParams(needs_layout_passes=False),
    out_type=jax.ShapeDtypeStruct((H, W), jnp.float32),
    scratch_types=[
        pltpu.VMEM((TABLE,), jnp.float32),
        pltpu.VMEM((BR, W), jnp.int32),
        pltpu.VMEM((BR, W), jnp.float32),
    ],
)
def _superpixel_gather(table_hbm, segm_hbm, out_hbm, table_v, idx_v, out_v):
    wid = lax.axis_index("s") * NC + lax.axis_index("c")
    row0 = wid * ROWS_W
    pltpu.sync_copy(table_hbm, table_v)

    def blk_body(bi, carry):
        r = row0 + bi * BR
        pltpu.sync_copy(segm_hbm.at[pl.ds(r, BR), :], idx_v)

        def gstep(i, c):
            for u in range(BR):
                s = i * L
                idx = idx_v[u, pl.ds(s, L)]
                out_v[u, pl.ds(s, L)] = plsc.load_gather(table_v, [idx])
            return c

        lax.fori_loop(0, W // L, gstep, 0)
        pltpu.sync_copy(out_v, out_hbm.at[pl.ds(r, BR), :])
        return carry

    lax.fori_loop(0, NBLK, blk_body, 0)


def kernel(input, segm_img):
    return _superpixel_gather(input, segm_img)


# speedup vs baseline: 965.8522x; 1.9595x over previous
import functools

import jax
import jax.numpy as jnp
from jax import lax
from jax.experimental import pallas as pl
from jax.experimental.pallas import tpu as pltpu
from jax.experimental.pallas import tpu_sc as plsc

H = 2048
W = 2048
TABLE = 10000

_info = plsc.get_sparse_core_info()
NC = _info.num_cores
NS = _info.num_subcores
L = _info.num_lanes
NW = NC * NS
ROWS_W = H // NW
BR = 8
NBLK = ROWS_W // BR
GCOLS = 2

_mesh = plsc.VectorSubcoreMesh(core_axis_name="c", subcore_axis_name="s")


@functools.partial(
    pl.kernel,
    mesh=_mesh,
    compiler_params=pltpu.CompilerParams(needs_layout_passes=False),
    out_type=jax.ShapeDtypeStruct((H, W), jnp.float32),
    scratch_types=[
        pltpu.VMEM((TABLE,), jnp.float32),
        pltpu.VMEM((2, BR, W), jnp.int32),
        pltpu.VMEM((2, BR, W), jnp.float32),
        pltpu.SemaphoreType.DMA,
        pltpu.SemaphoreType.DMA,
        pltpu.SemaphoreType.DMA,
        pltpu.SemaphoreType.DMA,
    ],
)
def _superpixel_gather(table_hbm, segm_hbm, out_hbm, table_v, idx_v, out_v,
                       sem_i0, sem_i1, sem_o0, sem_o1):
    sem_i = (sem_i0, sem_i1)
    sem_o = (sem_o0, sem_o1)
    wid = lax.axis_index("s") * NC + lax.axis_index("c")
    row0 = wid * ROWS_W
    pltpu.sync_copy(table_hbm, table_v)

    def start_in(bi):
        r = row0 + bi * BR
        return pltpu.async_copy(
            segm_hbm.at[pl.ds(r, BR), :], idx_v.at[bi % 2], sem_i[bi % 2])

    def start_out(bi):
        r = row0 + bi * BR
        return pltpu.async_copy(
            out_v.at[bi % 2], out_hbm.at[pl.ds(r, BR), :], sem_o[bi % 2])

    in_copy = start_in(0)
    out_copies = [None, None]
    for bi in range(NBLK):
        next_in = start_in(bi + 1) if bi + 1 < NBLK else None
        in_copy.wait()
        if out_copies[bi % 2] is not None:
            out_copies[bi % 2].wait()
        ib = idx_v.at[bi % 2]
        ob = out_v.at[bi % 2]

        def gstep(i, c, ib=ib, ob=ob):
            pos = [(u, (i * GCOLS + g) * L)
                   for g in range(GCOLS) for u in range(BR)]
            vecs = [ib[u, pl.ds(s, L)] for (u, s) in pos]
            vals = [plsc.load_gather(table_v, [v]) for v in vecs]
            for (u, s), val in zip(pos, vals):
                ob[u, pl.ds(s, L)] = val
            return c

        lax.fori_loop(0, W // (GCOLS * L), gstep, 0)
        out_copies[bi % 2] = start_out(bi)
        in_copy = next_in
    out_copies[0].wait()
    out_copies[1].wait()


def kernel(input, segm_img):
    return _superpixel_gather(input, segm_img)


# speedup vs baseline: 1023.3188x; 1.0595x over previous
import functools

import jax
import jax.numpy as jnp
from jax import lax
from jax.experimental import pallas as pl
from jax.experimental.pallas import tpu as pltpu
from jax.experimental.pallas import tpu_sc as plsc

H = 2048
W = 2048
TABLE = 10000

_info = plsc.get_sparse_core_info()
NC = _info.num_cores
NS = _info.num_subcores
L = _info.num_lanes
NW = NC * NS
ROWS_W = H // NW
BR = 8
NBLK = ROWS_W // BR
GCOLS = 2
NPOS = [(u, g) for g in range(GCOLS) for u in range(BR)]
STEPS = W // (GCOLS * L)

_mesh = plsc.VectorSubcoreMesh(core_axis_name="c", subcore_axis_name="s")


@functools.partial(
    pl.kernel,
    mesh=_mesh,
    compiler_params=pltpu.CompilerParams(needs_layout_passes=False),
    out_type=jax.ShapeDtypeStruct((H, W), jnp.float32),
    scratch_types=[
        pltpu.VMEM((TABLE,), jnp.float32),
        pltpu.VMEM((2, BR, W), jnp.int32),
        pltpu.VMEM((2, BR, W), jnp.float32),
        pltpu.SemaphoreType.DMA,
        pltpu.SemaphoreType.DMA,
        pltpu.SemaphoreType.DMA,
        pltpu.SemaphoreType.DMA,
    ],
)
def _superpixel_gather(table_hbm, segm_hbm, out_hbm, table_v, idx_v, out_v,
                       sem_i0, sem_i1, sem_o0, sem_o1):
    sem_i = (sem_i0, sem_i1)
    sem_o = (sem_o0, sem_o1)
    wid = lax.axis_index("s") * NC + lax.axis_index("c")
    row0 = wid * ROWS_W
    pltpu.sync_copy(table_hbm, table_v)

    def start_in(bi):
        r = row0 + bi * BR
        return pltpu.async_copy(
            segm_hbm.at[pl.ds(r, BR), :], idx_v.at[bi % 2], sem_i[bi % 2])

    def start_out(bi):
        r = row0 + bi * BR
        return pltpu.async_copy(
            out_v.at[bi % 2], out_hbm.at[pl.ds(r, BR), :], sem_o[bi % 2])

    in_copy = start_in(0)
    out_copies = [None, None]
    for bi in range(NBLK):
        next_in = start_in(bi + 1) if bi + 1 < NBLK else None
        in_copy.wait()
        if out_copies[bi % 2] is not None:
            out_copies[bi % 2].wait()
        ib = idx_v.at[bi % 2]
        ob = out_v.at[bi % 2]

        @plsc.parallel_loop(0, STEPS, 1, unroll=2)
        def _gather_groups(i, ib=ib, ob=ob):
            vecs = [ib[u, pl.ds((i * GCOLS + g) * L, L)] for (u, g) in NPOS]
            vals = [plsc.load_gather(table_v, [v]) for v in vecs]
            for (u, g), val in zip(NPOS, vals):
                ob[u, pl.ds((i * GCOLS + g) * L, L)] = val
        out_copies[bi % 2] = start_out(bi)
        in_copy = next_in
    out_copies[0].wait()
    out_copies[1].wait()


def kernel(input, segm_img):
    return _superpixel_gather(input, segm_img)


# speedup vs baseline: 1048.2525x; 1.0244x over previous
import functools

import jax
import jax.numpy as jnp
from jax import lax
from jax.experimental import pallas as pl
from jax.experimental.pallas import tpu as pltpu
from jax.experimental.pallas import tpu_sc as plsc

H = 2048
W = 2048
TABLE = 10000

_info = plsc.get_sparse_core_info()
NC = _info.num_cores
NS = _info.num_subcores
L = _info.num_lanes
NW = NC * NS
ROWS_W = H // NW
BR = 8
NBLK = ROWS_W // BR
GCOLS = 2
NPOS = [(u, g) for g in range(GCOLS) for u in range(BR)]
STEPS = W // (GCOLS * L)

_mesh = plsc.VectorSubcoreMesh(core_axis_name="c", subcore_axis_name="s")


@functools.partial(
    pl.kernel,
    mesh=_mesh,
    compiler_params=pltpu.CompilerParams(needs_layout_passes=False),
    out_type=jax.ShapeDtypeStruct((H, W), jnp.float32),
    scratch_types=[
        pltpu.VMEM((TABLE,), jnp.float32),
        pltpu.VMEM((2, BR, W), jnp.int32),
        pltpu.VMEM((2, BR, W), jnp.float32),
        pltpu.SemaphoreType.DMA,
        pltpu.SemaphoreType.DMA,
        pltpu.SemaphoreType.DMA,
        pltpu.SemaphoreType.DMA,
    ],
)
def _superpixel_gather(table_hbm, segm_hbm, out_hbm, table_v, idx_v, out_v,
                       sem_i0, sem_i1, sem_o0, sem_o1):
    sem_i = (sem_i0, sem_i1)
    sem_o = (sem_o0, sem_o1)
    wid = lax.axis_index("s") * NC + lax.axis_index("c")
    row0 = wid * ROWS_W

    def start_in(bi):
        r = row0 + bi * BR
        return pltpu.async_copy(
            segm_hbm.at[pl.ds(r, BR), :], idx_v.at[bi % 2], sem_i[bi % 2])

    def start_out(bi):
        r = row0 + bi * BR
        return pltpu.async_copy(
            out_v.at[bi % 2], out_hbm.at[pl.ds(r, BR), :], sem_o[bi % 2])

    in_copy = start_in(0)
    pltpu.sync_copy(table_hbm, table_v)
    out_copies = [None, None]
    for bi in range(NBLK):
        next_in = start_in(bi + 1) if bi + 1 < NBLK else None
        in_copy.wait()
        if out_copies[bi % 2] is not None:
            out_copies[bi % 2].wait()
        ib = idx_v.at[bi % 2]
        ob = out_v.at[bi % 2]

        @plsc.parallel_loop(0, STEPS, 1, unroll=2)
        def _gather_groups(i, ib=ib, ob=ob):
            vecs = [ib[u, pl.ds((i * GCOLS + g) * L, L)] for (u, g) in NPOS]
            vals = [plsc.load_gather(table_v, [v]) for v in vecs]
            for (u, g), val in zip(NPOS, vals):
                ob[u, pl.ds((i * GCOLS + g) * L, L)] = val
        out_copies[bi % 2] = start_out(bi)
        in_copy = next_in
    out_copies[0].wait()
    out_copies[1].wait()


def kernel(input, segm_img):
    return _superpixel_gather(input, segm_img)
